# Initial kernel scaffold; baseline (speedup 1.0000x reference)
import jax
import jax.numpy as jnp
from jax.experimental import pallas as pl


def kernel(x, edge_index, W, att_src, att_dst, bias, fc_W, fc_b):
    raise NotImplementedError("write your pallas kernel here")


# speedup vs baseline: 8.7286x; 8.7286x over previous
import jax
import jax.numpy as jnp
from jax import lax
from jax.experimental import pallas as pl
from jax.experimental.pallas import tpu as pltpu
from jax.experimental.pallas import tpu_sc as plsc

_N = 10000
_E = 320000
_F = 128
_NC = 2
_NS = 16
_NW = _NC * _NS
_C = 64
_K = ((_E + _NS * _C - 1) // (_NS * _C)) * _C
_CH = _K // _C
_EP = _K * _NS
_RH = 5120
_RPS = _RH // _NS
_DR = 80
_BLK = 2000


def _sc_body(h_hbm, asrc_hbm, adst_hbm, edge_hbm,
             out_hbm, outden_hbm,
             asrc_v, adst_v, edgeb, srcb2, dstb2, wb,
             rows_in, rows_out, denom_v, zb, rowidx, accd, acc, sem):
    c = lax.axis_index("c")
    s = lax.axis_index("s")
    lo = c * _RH

    pltpu.sync_copy(asrc_hbm, asrc_v)
    pltpu.sync_copy(adst_hbm, adst_v)

    def _zden(j, carry):
        for k in range(8):
            denom_v[j, 16 * k:16 * (k + 1)] = jnp.zeros((16,), jnp.float32)
        return carry
    lax.fori_loop(0, _DR, _zden, 0)
    lanes = lax.broadcasted_iota(jnp.int32, (16,), 0)
    for i in range(_DR // 16):
        rowidx[16 * i:16 * (i + 1)] = lanes + 16 * i

    def _zrow(j, carry):
        for k in range(_F // 16):
            zb[j, 16 * k:16 * (k + 1)] = jnp.zeros((16,), jnp.float32)
        return carry
    lax.fori_loop(0, _RPS, _zrow, 0)
    pltpu.sync_copy(zb, acc.at[pl.ds(s * _RPS, _RPS)])

    @pl.when(s == 0)
    def _():
        pltpu.sync_copy(zb.at[pl.ds(0, _DR)], accd)
    plsc.subcore_barrier()

    def _chunk(t, carry):
        base = s * _K + t * _C
        pltpu.sync_copy(edge_hbm.at[pl.ds(base, _C)], edgeb)
        for i in range(_C // 16):
            pv = edgeb[16 * i:16 * (i + 1)]
            sv = lax.shift_right_logical(pv, 14)
            dv = lax.bitwise_and(pv, 16383)
            e = plsc.load_gather(asrc_v, [sv]) + plsc.load_gather(adst_v, [dv])
            e = jnp.maximum(e, 0.0) + 0.2 * jnp.minimum(e, 0.0)
            w = jnp.exp(e)
            wb[16 * i:16 * (i + 1)] = w
            local = dv - lo
            keep = jnp.logical_and(
                jnp.logical_and(local >= 0, local < _RH),
                base + 16 * i + lanes < _E)
            srcb2[16 * i:16 * (i + 1)] = jnp.where(keep, sv, -1)
            dstb2[16 * i:16 * (i + 1)] = jnp.where(keep, local, -1)
            plsc.addupdate_scatter(
                denom_v, [lax.shift_right_logical(dv, 7),
                          lax.bitwise_and(dv, 127)], w, mask=keep)
        pltpu.async_copy(
            h_hbm.at[plsc.Indices(srcb2, ignored_value=-1)],
            rows_in, sem).wait()

        def _row(j, rcarry):
            w = plsc.load_gather(wb, [jnp.full((16,), j, jnp.int32)])
            for k in range(_F // 16):
                rows_out[j, 16 * k:16 * (k + 1)] = (
                    rows_in[j, 16 * k:16 * (k + 1)] * w)
            return rcarry
        lax.fori_loop(0, _C, _row, 0)

        pltpu.sync_copy(rows_out,
                        acc.at[plsc.Indices(dstb2, ignored_value=-1)],
                        add=True)
        return carry
    lax.fori_loop(0, _CH, _chunk, 0)

    pltpu.sync_copy(denom_v, accd.at[rowidx], add=True)
    plsc.subcore_barrier()
    pltpu.sync_copy(acc.at[pl.ds(s * _RPS, _RPS)], zb)
    pltpu.sync_copy(zb, out_hbm.at[c, pl.ds(s * _RPS, _RPS)])

    @pl.when(s == 0)
    def _():
        pltpu.sync_copy(accd, outden_hbm.at[c])


import functools


@functools.cache
def _make_sc_call():
  return pl.kernel(
    _sc_body,
    out_type=[
        jax.ShapeDtypeStruct((_NC, _RH, _F), jnp.float32),
        jax.ShapeDtypeStruct((_NC, _DR, 128), jnp.float32),
    ],
    mesh=plsc.VectorSubcoreMesh(core_axis_name="c", subcore_axis_name="s",
                                num_cores=_NC, num_subcores=_NS),
    scratch_types=[
        pltpu.VMEM((_N,), jnp.float32),
        pltpu.VMEM((_N,), jnp.float32),
        pltpu.VMEM((_C,), jnp.int32),
        pltpu.VMEM((_C,), jnp.int32),
        pltpu.VMEM((_C,), jnp.int32),
        pltpu.VMEM((_C,), jnp.float32),
        pltpu.VMEM((_C, _F), jnp.float32),
        pltpu.VMEM((_C, _F), jnp.float32),
        pltpu.VMEM((_DR, 128), jnp.float32),
        pltpu.VMEM((_RPS, _F), jnp.float32),
        pltpu.VMEM((_DR,), jnp.int32),
        pltpu.VMEM_SHARED((_DR, 128), jnp.float32),
        pltpu.VMEM_SHARED((_RH, _F), jnp.float32),
        pltpu.SemaphoreType.DMA,
    ],
    compiler_params=pltpu.CompilerParams(needs_layout_passes=False),
  )


def _dense_body(x_ref, w_ref, as_ref, ad_ref, h_ref, a1_ref, a2_ref):
    h = jnp.dot(x_ref[...], w_ref[...], preferred_element_type=jnp.float32)
    h_ref[...] = h
    a1_ref[...] = jnp.sum(h * as_ref[...], axis=1, keepdims=True)
    a2_ref[...] = jnp.sum(h * ad_ref[...], axis=1, keepdims=True)


_dense_part = pl.pallas_call(
    _dense_body,
    grid=(_N // _BLK,),
    in_specs=[
        pl.BlockSpec((_BLK, _F), lambda i: (i, 0)),
        pl.BlockSpec((_F, _F), lambda i: (0, 0)),
        pl.BlockSpec((1, _F), lambda i: (0, 0)),
        pl.BlockSpec((1, _F), lambda i: (0, 0)),
    ],
    out_specs=[
        pl.BlockSpec((_BLK, _F), lambda i: (i, 0)),
        pl.BlockSpec((_BLK, 1), lambda i: (i, 0)),
        pl.BlockSpec((_BLK, 1), lambda i: (i, 0)),
    ],
    out_shape=[
        jax.ShapeDtypeStruct((_N, _F), jnp.float32),
        jax.ShapeDtypeStruct((_N, 1), jnp.float32),
        jax.ShapeDtypeStruct((_N, 1), jnp.float32),
    ],
)


def _head_body(n_ref, d_ref, h_ref, a1_ref, a2_ref,
               b_ref, fw_ref, fb_ref, o_ref):
    e = a1_ref[...] + a2_ref[...]
    w = jnp.exp(jnp.maximum(e, 0.0) + 0.2 * jnp.minimum(e, 0.0))
    num = n_ref[...] + w * h_ref[...]
    den = jnp.sum(d_ref[...], axis=0) + w + 1e-16
    out = num / den + b_ref[...]
    logits = jnp.dot(out, fw_ref[...], preferred_element_type=jnp.float32)
    logits = logits + fb_ref[...]
    m = jnp.max(logits, axis=1, keepdims=True)
    lse = m + jnp.log(jnp.sum(jnp.exp(logits - m), axis=1, keepdims=True))
    o_ref[...] = logits - lse


def _make_head(n_class):
    return pl.pallas_call(
        _head_body,
        grid=(_N // _BLK,),
        in_specs=[
            pl.BlockSpec((_BLK, _F), lambda i: (i, 0)),
            pl.BlockSpec((_NC, _BLK, 1), lambda i: (0, i, 0)),
            pl.BlockSpec((_BLK, _F), lambda i: (i, 0)),
            pl.BlockSpec((_BLK, 1), lambda i: (i, 0)),
            pl.BlockSpec((_BLK, 1), lambda i: (i, 0)),
            pl.BlockSpec((1, _F), lambda i: (0, 0)),
            pl.BlockSpec((_F, n_class), lambda i: (0, 0)),
            pl.BlockSpec((1, n_class), lambda i: (0, 0)),
        ],
        out_specs=pl.BlockSpec((_BLK, n_class), lambda i: (i, 0)),
        out_shape=jax.ShapeDtypeStruct((_N, n_class), jnp.float32),
    )


def kernel(x, edge_index, W, att_src, att_dst, bias, fc_W, fc_b):
    n_class = fc_W.shape[0]
    h, asrc, adst = _dense_part(x, W, att_src.reshape(1, _F),
                                att_dst.reshape(1, _F))

    src = edge_index[0].astype(jnp.int32)
    dst = edge_index[1].astype(jnp.int32)
    pad = _EP - _E
    packed = jnp.concatenate(
        [lax.shift_left(src, 14) | dst, jnp.zeros((pad,), jnp.int32)])

    accs, dens = _make_sc_call()(h, asrc.reshape(_N), adst.reshape(_N),
                                 packed)

    num = accs.reshape(_NC * _RH, _F)[:_N]
    den = dens.reshape(_NC, _DR * 128, 1)[:, :_N]

    return _make_head(n_class)(
        num, den, h, asrc, adst,
        bias.reshape(1, _F), fc_W.T, fc_b.reshape(1, n_class))


# speedup vs baseline: 26.1419x; 2.9950x over previous
import jax
import jax.numpy as jnp
from jax import lax
from jax.experimental import pallas as pl
from jax.experimental.pallas import tpu as pltpu
from jax.experimental.pallas import tpu_sc as plsc

_N = 10000
_E = 320000
_F = 128
_NC = 2
_NS = 16
_NW = _NC * _NS
_C = 48
_CH = -2 * (-(_E // (_NS * _C) + 1) // 2)
_K = _CH * _C
_EP = _K * _NS
_RH = 5008
_RPS = 320
_RPL = _RH - 15 * _RPS
_DR = 80
_BLK = 2000


def _sc_body(h_hbm, asrc_hbm, adst_hbm, edge_hbm,
             out_hbm, outden_hbm,
             asrc_v, adst_v, edgeb0, edgeb1, srcb0, srcb1, dstb0, dstb1,
             wb0, wb1, rin0, rin1, rows_out, denom_v, zb, dstb3, acc,
             sem0, sem1, esem0, esem1, ssem):
    edgeb = [edgeb0, edgeb1]
    srcb2 = [srcb0, srcb1]
    dstb2 = [dstb0, dstb1]
    wb = [wb0, wb1]
    rows_in = [rin0, rin1]
    sem = [sem0, sem1]
    esem = [esem0, esem1]
    c = lax.axis_index("c")
    s = lax.axis_index("s")
    lo = c * _RH

    pltpu.sync_copy(asrc_hbm, asrc_v)
    pltpu.sync_copy(adst_hbm, adst_v)

    def _zden(j, carry):
        for k in range(8):
            denom_v[j, 16 * k:16 * (k + 1)] = jnp.zeros((16,), jnp.float32)
        return carry
    lax.fori_loop(0, _DR, _zden, 0)
    lanes = lax.broadcasted_iota(jnp.int32, (16,), 0)

    def _zrow(j, carry):
        for k in range(_F // 16):
            zb[j, 16 * k:16 * (k + 1)] = jnp.zeros((16,), jnp.float32)
        return carry
    lax.fori_loop(0, _RPS, _zrow, 0)

    @pl.when(s < _NS - 1)
    def _():
        pltpu.sync_copy(zb, acc.at[pl.ds(s * _RPS, _RPS)])

    @pl.when(s == _NS - 1)
    def _():
        pltpu.sync_copy(zb.at[pl.ds(0, _RPL)],
                        acc.at[pl.ds((_NS - 1) * _RPS, _RPL)])

    plsc.subcore_barrier()

    neg1 = jnp.full((16,), -1, jnp.int32)

    def _weights(t, b):
        base = s * _K + t * _C
        for i in range(_C // 16):
            srcb2[b][16 * i:16 * (i + 1)] = neg1
            dstb2[b][16 * i:16 * (i + 1)] = neg1
        cnt = jnp.int32(0)
        for i in range(_C // 16):
            pv = edgeb[b][16 * i:16 * (i + 1)]
            sv = lax.shift_right_logical(pv, 14)
            dv = lax.bitwise_and(pv, 16383)
            e = plsc.load_gather(asrc_v, [sv]) + plsc.load_gather(adst_v, [dv])
            e = jnp.maximum(e, 0.0) + 0.2 * jnp.minimum(e, 0.0)
            w = jnp.exp(e)
            local = dv - lo
            idv = base + 16 * i + lanes
            keep = jnp.logical_and(
                jnp.logical_and(local >= 0, local < _RH),
                jnp.logical_and(idv < _E, idv < s * _K + _K))
            plsc.addupdate_scatter(
                denom_v, [lax.shift_right_logical(dv, 7),
                          lax.bitwise_and(dv, 127)], w, mask=keep)
            plsc.store_compressed(srcb2[b].at[pl.ds(cnt, 16)], sv, mask=keep)
            plsc.store_compressed(dstb2[b].at[pl.ds(cnt, 16)], local, mask=keep)
            plsc.store_compressed(wb[b].at[pl.ds(cnt, 16)], w, mask=keep)
            cnt = cnt + jnp.sum(keep.astype(jnp.int32))
        return cnt

    def _issue_gather(b):
        return pltpu.async_copy(
            h_hbm.at[plsc.Indices(srcb2[b], ignored_value=-1)],
            rows_in[b], sem[b])

    def _issue_scatter():
        return pltpu.async_copy(
            rows_out, acc.at[plsc.Indices(dstb3, ignored_value=-1)],
            ssem, add=True)

    pltpu.sync_copy(edge_hbm.at[pl.ds(s * _K, _C)], edgeb[0])
    cnt0 = _weights(0, 0)
    _issue_gather(0)

    def _pair(q, carry):
        cnts = list(carry)
        for b in range(2):
            t = 2 * q + b
            nb = 1 - b
            pltpu.async_copy(
                edge_hbm.at[pl.ds(s * _K + (t + 1) * _C, _C)],
                edgeb[nb], esem[nb])
            pltpu.make_async_copy(
                h_hbm.at[plsc.Indices(srcb2[b], ignored_value=-1)],
                rows_in[b], sem[b]).wait()
            pltpu.make_async_copy(
                edge_hbm.at[pl.ds(s * _K + (t + 1) * _C, _C)],
                edgeb[nb], esem[nb]).wait()
            cnts[nb] = _weights(t + 1, nb)

            @pl.when(t + 1 < _CH)
            def _():
                _issue_gather(nb)

            @pl.when(t > 0)
            def _():
                pltpu.make_async_copy(
                    rows_out,
                    acc.at[plsc.Indices(dstb3, ignored_value=-1)],
                    ssem).wait()

            cnt8 = lax.bitwise_and(cnts[b] + 7, -8)

            @plsc.parallel_loop(0, cnt8, 1, unroll=8)
            def _row(j):
                w = plsc.load_gather(wb[b], [jnp.full((16,), j, jnp.int32)])
                for k in range(_F // 16):
                    rows_out[j, 16 * k:16 * (k + 1)] = (
                        rows_in[b][j, 16 * k:16 * (k + 1)] * w)

            for i in range(_C // 16):
                dstb3[16 * i:16 * (i + 1)] = dstb2[b][16 * i:16 * (i + 1)]
            _issue_scatter()
        return tuple(cnts)
    lax.fori_loop(0, _CH // 2, _pair, (cnt0, jnp.int32(0)))
    pltpu.make_async_copy(
        rows_out, acc.at[plsc.Indices(dstb3, ignored_value=-1)],
        ssem).wait()

    plsc.subcore_barrier()

    @pl.when(s < _NS - 1)
    def _():
        pltpu.sync_copy(acc.at[pl.ds(s * _RPS, _RPS)],
                        out_hbm.at[c, pl.ds(s * _RPS, _RPS)])

    @pl.when(s == _NS - 1)
    def _():
        pltpu.sync_copy(acc.at[pl.ds((_NS - 1) * _RPS, _RPL)],
                        out_hbm.at[c, pl.ds((_NS - 1) * _RPS, _RPL)])

    pltpu.sync_copy(denom_v, outden_hbm.at[s * _NC + c])


import functools


@functools.cache
def _make_sc_call():
  return pl.kernel(
    _sc_body,
    out_type=[
        jax.ShapeDtypeStruct((_NC, _RH, _F), jnp.float32),
        jax.ShapeDtypeStruct((_NW, _DR, 128), jnp.float32),
    ],
    mesh=plsc.VectorSubcoreMesh(core_axis_name="c", subcore_axis_name="s",
                                num_cores=_NC, num_subcores=_NS),
    scratch_types=[
        pltpu.VMEM((_N,), jnp.float32),
        pltpu.VMEM((_N,), jnp.float32),
        pltpu.VMEM((_C,), jnp.int32),
        pltpu.VMEM((_C,), jnp.int32),
        pltpu.VMEM((_C,), jnp.int32),
        pltpu.VMEM((_C,), jnp.int32),
        pltpu.VMEM((_C,), jnp.int32),
        pltpu.VMEM((_C,), jnp.int32),
        pltpu.VMEM((_C,), jnp.float32),
        pltpu.VMEM((_C,), jnp.float32),
        pltpu.VMEM((_C, _F), jnp.float32),
        pltpu.VMEM((_C, _F), jnp.float32),
        pltpu.VMEM((_C, _F), jnp.float32),
        pltpu.VMEM((_DR, 128), jnp.float32),
        pltpu.VMEM((_RPS, _F), jnp.float32),
        pltpu.VMEM((_C,), jnp.int32),
        pltpu.VMEM_SHARED((_RH, _F), jnp.float32),
        pltpu.SemaphoreType.DMA,
        pltpu.SemaphoreType.DMA,
        pltpu.SemaphoreType.DMA,
        pltpu.SemaphoreType.DMA,
        pltpu.SemaphoreType.DMA,
    ],
    compiler_params=pltpu.CompilerParams(needs_layout_passes=False),
  )


def _dense_body(x_ref, w_ref, as_ref, ad_ref, h_ref, a1_ref, a2_ref):
    h = jnp.dot(x_ref[...], w_ref[...], preferred_element_type=jnp.float32)
    h_ref[...] = h
    a1_ref[...] = jnp.sum(h * as_ref[...], axis=1, keepdims=True)
    a2_ref[...] = jnp.sum(h * ad_ref[...], axis=1, keepdims=True)


_dense_part = pl.pallas_call(
    _dense_body,
    grid=(_N // _BLK,),
    in_specs=[
        pl.BlockSpec((_BLK, _F), lambda i: (i, 0)),
        pl.BlockSpec((_F, _F), lambda i: (0, 0)),
        pl.BlockSpec((1, _F), lambda i: (0, 0)),
        pl.BlockSpec((1, _F), lambda i: (0, 0)),
    ],
    out_specs=[
        pl.BlockSpec((_BLK, _F), lambda i: (i, 0)),
        pl.BlockSpec((_BLK, 1), lambda i: (i, 0)),
        pl.BlockSpec((_BLK, 1), lambda i: (i, 0)),
    ],
    out_shape=[
        jax.ShapeDtypeStruct((_N, _F), jnp.float32),
        jax.ShapeDtypeStruct((_N, 1), jnp.float32),
        jax.ShapeDtypeStruct((_N, 1), jnp.float32),
    ],
)


def _densum_body(d_ref, o_ref):
    o_ref[...] = jnp.sum(d_ref[...], axis=0)


_densum = pl.pallas_call(
    _densum_body,
    grid=(1,),
    in_specs=[pl.BlockSpec((_NW, _DR, 128), lambda i: (0, 0, 0))],
    out_specs=pl.BlockSpec((_DR, 128), lambda i: (0, 0)),
    out_shape=jax.ShapeDtypeStruct((_DR, 128), jnp.float32),
)


def _head_body(n_ref, d_ref, h_ref, a1_ref, a2_ref,
               b_ref, fw_ref, fb_ref, o_ref):
    e = a1_ref[...] + a2_ref[...]
    w = jnp.exp(jnp.maximum(e, 0.0) + 0.2 * jnp.minimum(e, 0.0))
    num = n_ref[...] + w * h_ref[...]
    den = d_ref[...] + w + 1e-16
    out = num / den + b_ref[...]
    logits = jnp.dot(out, fw_ref[...], preferred_element_type=jnp.float32)
    logits = logits + fb_ref[...]
    m = jnp.max(logits, axis=1, keepdims=True)
    lse = m + jnp.log(jnp.sum(jnp.exp(logits - m), axis=1, keepdims=True))
    o_ref[...] = logits - lse


def _make_head(n_class):
    return pl.pallas_call(
        _head_body,
        grid=(_N // _BLK,),
        in_specs=[
            pl.BlockSpec((_BLK, _F), lambda i: (i, 0)),
            pl.BlockSpec((_BLK, 1), lambda i: (i, 0)),
            pl.BlockSpec((_BLK, _F), lambda i: (i, 0)),
            pl.BlockSpec((_BLK, 1), lambda i: (i, 0)),
            pl.BlockSpec((_BLK, 1), lambda i: (i, 0)),
            pl.BlockSpec((1, _F), lambda i: (0, 0)),
            pl.BlockSpec((_F, n_class), lambda i: (0, 0)),
            pl.BlockSpec((1, n_class), lambda i: (0, 0)),
        ],
        out_specs=pl.BlockSpec((_BLK, n_class), lambda i: (i, 0)),
        out_shape=jax.ShapeDtypeStruct((_N, n_class), jnp.float32),
    )


def kernel(x, edge_index, W, att_src, att_dst, bias, fc_W, fc_b):
    n_class = fc_W.shape[0]
    h, asrc, adst = _dense_part(x, W, att_src.reshape(1, _F),
                                att_dst.reshape(1, _F))

    src = edge_index[0].astype(jnp.int32)
    dst = edge_index[1].astype(jnp.int32)
    pad = _EP - _E + _C
    packed = jnp.concatenate(
        [lax.shift_left(src, 14) | dst, jnp.zeros((pad,), jnp.int32)])

    accs, dens = _make_sc_call()(h, asrc.reshape(_N), adst.reshape(_N),
                                 packed)

    num = accs.reshape(_NC * _RH, _F)[:_N]
    den = _densum(dens).reshape(_DR * 128, 1)[:_N]

    return _make_head(n_class)(
        num, den, h, asrc, adst,
        bias.reshape(1, _F), fc_W.T, fc_b.reshape(1, n_class))
